# baseline (speedup 1.0000x reference)
import math

import jax
import jax.numpy as jnp
from jax.experimental import pallas as pl
from jax.experimental.pallas import tpu as pltpu
from jax.experimental.pallas import tpu_sc as plsc

_D_MODEL = 64
_SCALE = math.sqrt(_D_MODEL)
_LANES = 16
_ROWS_PER_STEP = 2


def _sc_embed(lut, x, b, s):
    mesh = plsc.VectorSubcoreMesh(core_axis_name="c", subcore_axis_name="s")

    @pl.kernel(
        out_type=jax.ShapeDtypeStruct((b, s, _D_MODEL), jnp.float32),
        mesh=mesh,
        compiler_params=pltpu.CompilerParams(use_tc_tiling_on_sc=False),
    )
    def k(lut_hbm, x_hbm, out_hbm):
        def body(idx_vmem, out_vmem):
            for r in range(_ROWS_PER_STEP):
                pltpu.sync_copy(lut_hbm.at[idx_vmem.at[r]], out_vmem.at[r])

            for r in range(_ROWS_PER_STEP):
                ov = out_vmem.at[r]

                @pl.loop(0, s, step=2)
                def _(i):
                    for j in range(2):
                        for c in range(_D_MODEL // _LANES):
                            slc = (pl.ds(i + j, 1), pl.ds(c * _LANES, _LANES))
                            ov.at[slc][...] = ov.at[slc][...] * _SCALE

        pltpu.emit_pipeline(
            body,
            grid=(b // _ROWS_PER_STEP,),
            in_specs=[pl.BlockSpec((_ROWS_PER_STEP, s), lambda i: (i, 0))],
            out_specs=[
                pl.BlockSpec((_ROWS_PER_STEP, s, _D_MODEL), lambda i: (i, 0, 0))
            ],
            core_axis_name=("c", "s"),
            dimension_semantics=(pltpu.PARALLEL,),
        )(x_hbm, out_hbm)

    return k(lut, x)


def kernel(x, lut):
    b, s = x.shape
    return _sc_embed(lut, x, b, s)


# speedup vs baseline: 1.4594x; 1.4594x over previous
import math

import jax
import jax.numpy as jnp
from jax import lax
from jax.experimental import pallas as pl
from jax.experimental.pallas import tpu as pltpu
from jax.experimental.pallas import tpu_sc as plsc

_D_MODEL = 64
_SCALE = math.sqrt(_D_MODEL)
_LANES = 16
_NC, _NS = 2, 16
_NW = _NC * _NS
_NBUF = 4


def _scale_chunk(src, dst, s):
    sv = src.at[0]
    dv = dst.at[0]

    @pl.loop(0, s, step=2)
    def _(r):
        for j in range(2):
            for c in range(_D_MODEL // _LANES):
                slc = (pl.ds(r + j, 1), pl.ds(c * _LANES, _LANES))
                dv.at[slc][...] = sv.at[slc][...] * _SCALE


def _sc_embed(lut, x, b, s):
    bpw = b // _NW
    nround = bpw // _NBUF
    assert bpw * _NW == b and nround * _NBUF == bpw and nround >= 3
    mesh = plsc.VectorSubcoreMesh(core_axis_name="c", subcore_axis_name="s")

    @pl.kernel(
        out_type=jax.ShapeDtypeStruct((b, s, _D_MODEL), jnp.float32),
        mesh=mesh,
        compiler_params=pltpu.CompilerParams(use_tc_tiling_on_sc=False),
        scratch_types=[
            pltpu.VMEM((bpw, s), jnp.int32),
            pltpu.VMEM((_NBUF, 1, s, _D_MODEL), jnp.float32),
            pltpu.VMEM((2, 1, s, _D_MODEL), jnp.float32),
            pltpu.SemaphoreType.DMA((_NBUF,)),
            pltpu.SemaphoreType.DMA((2,)),
            pltpu.SemaphoreType.DMA,
        ],
    )
    def k(lut_hbm, x_hbm, out_hbm, idx_v, rows, sbuf, gsem, ssem, isem):
        wid = lax.axis_index("s") * _NC + lax.axis_index("c")
        base = wid * bpw
        pltpu.async_copy(x_hbm.at[pl.ds(base, bpw)], idx_v, isem).wait()

        def fire_gather(c, bi):
            pltpu.async_copy(
                lut_hbm.at[idx_v.at[c]], rows.at[bi, 0], gsem.at[bi]
            )

        def wait_gather(c, bi):
            pltpu.make_async_copy(
                lut_hbm.at[idx_v.at[c]], rows.at[bi, 0], gsem.at[bi]
            ).wait()

        def fire_store(c, sb):
            pltpu.async_copy(
                sbuf.at[sb], out_hbm.at[pl.ds(base + c, 1)], ssem.at[sb]
            )

        def wait_store(c, sb):
            pltpu.make_async_copy(
                sbuf.at[sb], out_hbm.at[pl.ds(base + c, 1)], ssem.at[sb]
            ).wait()

        for bi in range(_NBUF):
            fire_gather(bi, bi)

        for bi in range(_NBUF):
            wait_gather(bi, bi)
            if bi >= 2:
                wait_store(bi - 2, bi % 2)
            _scale_chunk(rows.at[bi], sbuf.at[bi % 2], s)
            fire_gather(_NBUF + bi, bi)
            fire_store(bi, bi % 2)

        @pl.loop(1, nround - 1)
        def _(r):
            cb = r * _NBUF
            for bi in range(_NBUF):
                c = cb + bi
                wait_gather(c, bi)
                wait_store(c - 2, bi % 2)
                _scale_chunk(rows.at[bi], sbuf.at[bi % 2], s)
                fire_gather(c + _NBUF, bi)
                fire_store(c, bi % 2)

        cb = (nround - 1) * _NBUF
        for bi in range(_NBUF):
            c = cb + bi
            wait_gather(c, bi)
            wait_store(c - 2, bi % 2)
            _scale_chunk(rows.at[bi], sbuf.at[bi % 2], s)
            fire_store(c, bi % 2)

        wait_store(bpw - 2, (bpw - 2) % 2)
        wait_store(bpw - 1, (bpw - 1) % 2)

    return k(lut, x)


def kernel(x, lut):
    b, s = x.shape
    return _sc_embed(lut, x, b, s)
